# baseline (speedup 1.0000x reference)
import jax
import jax.numpy as jnp
from jax import lax
from jax.experimental import pallas as pl
from jax.experimental.pallas import tpu as pltpu
from jax.experimental.pallas import tpu_sc as plsc

N_TOKENS = 16384
D_MODEL = 2048
N_GROUPS = 8

_TN = 1024
_PIPE = 2
_NCHUNK = N_TOKENS // _PIPE

_NC = 2
_NS = 16
_NW = _NC * _NS
_TOK_PER_W = _NCHUNK // _NW
_LANES = 16
_CHUNKS = _TOK_PER_W // _LANES


def _logits_body(x_ref, w_ref, out_ref):
    out_ref[...] = lax.dot_general(
        w_ref[...], x_ref[...],
        dimension_numbers=(((1,), (1,)), ((), ())),
        preferred_element_type=jnp.float32,
    )


def _compute_logits_t(x, W, chunk):
    blocks = _NCHUNK // _TN
    return pl.pallas_call(
        _logits_body,
        grid=(blocks,),
        in_specs=[
            pl.BlockSpec((_TN, D_MODEL),
                         lambda i, c=chunk, b=blocks: (c * b + i, 0)),
            pl.BlockSpec((N_GROUPS, D_MODEL), lambda i: (0, 0)),
        ],
        out_specs=pl.BlockSpec((N_GROUPS, _TN), lambda i: (0, i)),
        out_shape=jax.ShapeDtypeStruct((N_GROUPS, _NCHUNK), jnp.float32),
    )(x, W)


def _route_body(logits_hbm, rw_hbm, mask_hbm, scores_hbm,
                lbuf, rwbuf, mbuf, sbuf):
    wid = lax.axis_index("s") * _NC + lax.axis_index("c")
    base = wid * _TOK_PER_W
    pltpu.sync_copy(logits_hbm.at[:, pl.ds(base, _TOK_PER_W)], lbuf)

    def chunk(c, carry):
        off = c * _LANES
        l = [lbuf[g, pl.ds(off, _LANES)] for g in range(N_GROUPS)]
        m = l[0]
        for g in range(1, N_GROUPS):
            m = jnp.maximum(m, l[g])
        e = [jnp.exp(v - m) for v in l]
        tot = e[0]
        for g in range(1, N_GROUPS):
            tot = tot + e[g]
        sc = [v / tot for v in e]
        ms = sc[0]
        for g in range(1, N_GROUPS):
            ms = jnp.maximum(ms, sc[g])
        rwv = ms / (ms + jnp.float32(1e-8))
        one = jnp.ones((_LANES,), jnp.float32)
        zero = jnp.zeros((_LANES,), jnp.float32)
        seen = jnp.zeros((_LANES,), jnp.bool_)
        for g in range(N_GROUPS):
            is_g = (sc[g] == ms) & jnp.logical_not(seen)
            seen = seen | is_g
            sbuf[g, pl.ds(off, _LANES)] = sc[g]
            mbuf[g, pl.ds(off, _LANES)] = jnp.where(is_g, one, zero)
            rwbuf[g, pl.ds(off, _LANES)] = jnp.where(is_g, rwv, zero)
        return carry

    lax.fori_loop(0, _CHUNKS, chunk, 0)
    dst = pl.ds(base, _TOK_PER_W)
    pltpu.sync_copy(rwbuf, rw_hbm.at[:, dst])
    pltpu.sync_copy(mbuf, mask_hbm.at[:, dst])
    pltpu.sync_copy(sbuf, scores_hbm.at[:, dst])


def _route(logits_t):
    mesh = plsc.VectorSubcoreMesh(core_axis_name="c", subcore_axis_name="s")
    out = jax.ShapeDtypeStruct((N_GROUPS, _NCHUNK), jnp.float32)
    f = pl.kernel(
        _route_body,
        out_type=[out, out, out],
        mesh=mesh,
        scratch_types=[pltpu.VMEM((N_GROUPS, _TOK_PER_W), jnp.float32)] * 4,
        compiler_params=pltpu.CompilerParams(needs_layout_passes=False),
    )
    return f(logits_t)


def kernel(x, W):
    pieces = []
    for c in range(_PIPE):
        logits_t = _compute_logits_t(x, W, c)
        pieces.append(_route(logits_t))
    rw, mask, scores = (jnp.concatenate(p, axis=1) for p in zip(*pieces))
    return rw.T, mask.T, scores.T


# speedup vs baseline: 1.1046x; 1.1046x over previous
import jax
import jax.numpy as jnp
from jax import lax
from jax.experimental import pallas as pl
from jax.experimental.pallas import tpu as pltpu
from jax.experimental.pallas import tpu_sc as plsc

N_TOKENS = 16384
D_MODEL = 2048
N_GROUPS = 8

_TN = 1024
_PIPE = 1
_NCHUNK = N_TOKENS // _PIPE

_NC = 2
_NS = 16
_NW = _NC * _NS
_TOK_PER_W = _NCHUNK // _NW
_LANES = 16
_CHUNKS = _TOK_PER_W // _LANES


def _logits_body(x_ref, w_ref, out_ref):
    out_ref[...] = lax.dot_general(
        w_ref[...], x_ref[...],
        dimension_numbers=(((1,), (1,)), ((), ())),
        preferred_element_type=jnp.float32,
    )


def _compute_logits_t(x, W, chunk):
    blocks = _NCHUNK // _TN
    return pl.pallas_call(
        _logits_body,
        grid=(blocks,),
        in_specs=[
            pl.BlockSpec((_TN, D_MODEL),
                         lambda i, c=chunk, b=blocks: (c * b + i, 0)),
            pl.BlockSpec((N_GROUPS, D_MODEL), lambda i: (0, 0)),
        ],
        out_specs=pl.BlockSpec((N_GROUPS, _TN), lambda i: (0, i)),
        out_shape=jax.ShapeDtypeStruct((N_GROUPS, _NCHUNK), jnp.float32),
    )(x, W)


def _route_body(logits_hbm, rw_hbm, mask_hbm, scores_hbm,
                lbuf, rwbuf, mbuf, sbuf, dsem):
    wid = lax.axis_index("s") * _NC + lax.axis_index("c")
    base = wid * _TOK_PER_W
    pltpu.sync_copy(logits_hbm.at[:, pl.ds(base, _TOK_PER_W)], lbuf)

    def chunk(c, carry):
        off = c * _LANES
        l = [lbuf[g, pl.ds(off, _LANES)] for g in range(N_GROUPS)]
        m = l[0]
        for g in range(1, N_GROUPS):
            m = jnp.maximum(m, l[g])
        e = [jnp.exp(v - m) for v in l]
        tot = e[0]
        for g in range(1, N_GROUPS):
            tot = tot + e[g]
        sc = [v / tot for v in e]
        ms = jnp.ones((_LANES,), jnp.float32) / tot
        rwv = ms / (ms + jnp.float32(1e-8))
        one = jnp.ones((_LANES,), jnp.float32)
        zero = jnp.zeros((_LANES,), jnp.float32)
        seen = jnp.zeros((_LANES,), jnp.bool_)
        for g in range(N_GROUPS):
            is_g = (sc[g] == ms) & jnp.logical_not(seen)
            seen = seen | is_g
            sbuf[g, pl.ds(off, _LANES)] = sc[g]
            mbuf[g, pl.ds(off, _LANES)] = jnp.where(is_g, one, zero)
            rwbuf[g, pl.ds(off, _LANES)] = jnp.where(is_g, rwv, zero)
        return carry

    lax.fori_loop(0, _CHUNKS, chunk, 0)
    dst = pl.ds(base, _TOK_PER_W)
    c1 = pltpu.async_copy(rwbuf, rw_hbm.at[:, dst], dsem)
    c2 = pltpu.async_copy(mbuf, mask_hbm.at[:, dst], dsem)
    c3 = pltpu.async_copy(sbuf, scores_hbm.at[:, dst], dsem)
    c1.wait()
    c2.wait()
    c3.wait()


def _route(logits_t):
    mesh = plsc.VectorSubcoreMesh(core_axis_name="c", subcore_axis_name="s")
    out = jax.ShapeDtypeStruct((N_GROUPS, _NCHUNK), jnp.float32)
    f = pl.kernel(
        _route_body,
        out_type=[out, out, out],
        mesh=mesh,
        scratch_types=[pltpu.VMEM((N_GROUPS, _TOK_PER_W), jnp.float32)] * 4
        + [pltpu.SemaphoreType.DMA],
        compiler_params=pltpu.CompilerParams(needs_layout_passes=False),
    )
    return f(logits_t)


def kernel(x, W):
    pieces = []
    for c in range(_PIPE):
        logits_t = _compute_logits_t(x, W, c)
        pieces.append(_route(logits_t))
    rw, mask, scores = (jnp.concatenate(p, axis=1) for p in zip(*pieces))
    return rw.T, mask.T, scores.T
